# baseline (device time: 994530 ns/iter reference)
import os

import jax

os.makedirs("/tmp/scband_jax_cache", exist_ok=True)
jax.config.update("jax_compilation_cache_dir", "/tmp/scband_jax_cache")
jax.config.update("jax_persistent_cache_min_compile_time_secs", 0)

import jax.numpy as jnp
from jax import lax
from jax.experimental import pallas as pl
from jax.experimental.pallas import tpu as pltpu

N_DEV = 4


def kernel(A, B):
    m_per, k = A.shape
    _, n = B.shape
    M = N_DEV * m_per
    half = m_per // 2

    TM = 512
    n_tiles = m_per // TM
    n_ctiles = n_tiles // 2
    n_atiles = n_tiles - n_ctiles
    piece = half // 2

    def body(a_ref, b_ref, out_ref, ag_ref, a_vmem, c_vmem, local_sem,
             sAR, rAR, sAL, rAL, sCR, rCR, sCL, rCL):
        my = lax.axis_index("i")
        left = (my + N_DEV - 1) % N_DEV
        right = (my + 1) % N_DEV
        diag = (my + 2) % N_DEV

        barrier = pltpu.get_barrier_semaphore()
        for nbr in (left, right):
            pl.semaphore_signal(
                barrier, inc=1, device_id=(nbr,),
                device_id_type=pl.DeviceIdType.MESH,
            )
        pl.semaphore_wait(barrier, 2)

        def rdma(src, dst, send_sem, recv_sem, dev):
            pltpu.make_async_remote_copy(
                src_ref=src, dst_ref=dst,
                send_sem=send_sem, recv_sem=recv_sem,
                device_id=(dev,), device_id_type=pl.DeviceIdType.MESH,
            ).start()

        def wait_recv(dst, recv_sem):
            pltpu.make_async_remote_copy(
                src_ref=dst, dst_ref=dst,
                send_sem=sAR.at[0], recv_sem=recv_sem,
                device_id=(right,), device_id_type=pl.DeviceIdType.MESH,
            ).wait_recv()

        def wait_send(src, send_sem):
            pltpu.make_async_remote_copy(
                src_ref=src, dst_ref=src,
                send_sem=send_sem, recv_sem=rAR.at[0],
                device_id=(right,), device_id_type=pl.DeviceIdType.MESH,
            ).wait_send()

        def matmul_tile(src_slice, out_rows):
            a_in = pltpu.make_async_copy(src_slice, a_vmem, local_sem)
            a_in.start()
            a_in.wait()
            c_vmem[...] = jnp.dot(
                a_vmem[...], b_ref[...], preferred_element_type=jnp.float32
            )
            c_out = pltpu.make_async_copy(
                c_vmem, out_ref.at[pl.ds(out_rows, TM), :], local_sem
            )
            c_out.start()
            c_out.wait()

        def own_tile(t):
            matmul_tile(a_ref.at[pl.ds(t * TM, TM), :], my * m_per + t * TM)

        def remote_tile(o, j):
            matmul_tile(
                ag_ref.at[o, pl.ds(j * TM, TM), :],
                o * m_per + half + j * TM,
            )

        def a_piece(o, p):
            return ag_ref.at[o, pl.ds(p * piece, piece), :]

        def c_tile(o, t):
            return out_ref.at[pl.ds(o * m_per + t * TM, TM), :]

        for p in range(2):
            src = a_ref.at[pl.ds(half + p * piece, piece), :]
            rdma(src, a_piece(my, p), sAR.at[p], rAR.at[p], right)
            rdma(src, a_piece(my, p), sAL.at[p], rAL.at[p], left)

        def own_c_tile(t, _):
            own_tile(t)
            rdma(c_tile(my, t), c_tile(my, t), sCR.at[t], rCR.at[t], right)
            rdma(c_tile(my, t), c_tile(my, t), sCL.at[t], rCL.at[t], left)
            return _

        lax.fori_loop(0, 2, own_c_tile, 0)

        wait_recv(a_piece(left, 0), rAR.at[0])
        rdma(a_piece(left, 0), a_piece(left, 0), sAR.at[2], rAR.at[2], right)

        lax.fori_loop(2, n_ctiles, own_c_tile, 0)

        wait_recv(a_piece(right, 1), rAL.at[1])
        rdma(a_piece(right, 1), a_piece(right, 1), sAL.at[2], rAL.at[2], left)

        wait_recv(a_piece(left, 1), rAR.at[1])
        lax.fori_loop(0, n_atiles, lambda j, _: (remote_tile(left, j), _)[1], 0)
        wait_recv(a_piece(right, 0), rAL.at[0])
        lax.fori_loop(0, n_atiles, lambda j, _: (remote_tile(right, j), _)[1], 0)

        for j in range(2):
            wait_recv(c_tile(left, j), rCR.at[j])
            rdma(c_tile(left, j), c_tile(left, j),
                 sCR.at[n_ctiles + j], rCR.at[n_ctiles + j], right)

        lax.fori_loop(n_ctiles, n_ctiles + 2, lambda t, _: (own_tile(t), _)[1], 0)
        wait_recv(c_tile(right, 2), rCL.at[2])
        rdma(c_tile(right, 2), c_tile(right, 2),
             sCL.at[n_ctiles], rCL.at[n_ctiles], left)
        lax.fori_loop(n_ctiles + 2, n_tiles, lambda t, _: (own_tile(t), _)[1], 0)
        wait_recv(c_tile(right, 3), rCL.at[3])
        rdma(c_tile(right, 3), c_tile(right, 3),
             sCL.at[n_ctiles + 1], rCL.at[n_ctiles + 1], left)

        wait_recv(a_piece(diag, 0), rAR.at[2])
        wait_recv(a_piece(diag, 1), rAL.at[2])
        lax.fori_loop(0, n_atiles, lambda j, _: (remote_tile(diag, j), _)[1], 0)

        wait_recv(c_tile(left, 2), rCR.at[2])
        wait_recv(c_tile(left, 3), rCR.at[3])
        wait_recv(c_tile(right, 0), rCL.at[0])
        wait_recv(c_tile(right, 1), rCL.at[1])
        wait_recv(c_tile(diag, 0), rCR.at[n_ctiles])
        wait_recv(c_tile(diag, 1), rCR.at[n_ctiles + 1])
        wait_recv(c_tile(diag, 2), rCL.at[n_ctiles])
        wait_recv(c_tile(diag, 3), rCL.at[n_ctiles + 1])

        for p in range(3):
            wait_send(a_piece(my, p % 2), sAR.at[p])
            wait_send(a_piece(my, p % 2), sAL.at[p])
        for t in range(n_ctiles + 2):
            wait_send(c_tile(my, t % n_ctiles), sCR.at[t])
            wait_send(c_tile(my, t % n_ctiles), sCL.at[t])

    return pl.pallas_call(
        body,
        out_shape=[
            jax.ShapeDtypeStruct((M, n), jnp.float32),
            jax.ShapeDtypeStruct((N_DEV, half, k), jnp.float32),
        ],
        in_specs=[
            pl.BlockSpec(memory_space=pl.ANY),
            pl.BlockSpec(memory_space=pltpu.VMEM),
        ],
        out_specs=[
            pl.BlockSpec(memory_space=pl.ANY),
            pl.BlockSpec(memory_space=pl.ANY),
        ],
        scratch_shapes=[
            pltpu.VMEM((TM, k), jnp.float32),
            pltpu.VMEM((TM, n), jnp.float32),
            pltpu.SemaphoreType.DMA,
            pltpu.SemaphoreType.DMA((3,)),
            pltpu.SemaphoreType.DMA((3,)),
            pltpu.SemaphoreType.DMA((3,)),
            pltpu.SemaphoreType.DMA((3,)),
            pltpu.SemaphoreType.DMA((6,)),
            pltpu.SemaphoreType.DMA((6,)),
            pltpu.SemaphoreType.DMA((6,)),
            pltpu.SemaphoreType.DMA((6,)),
        ],
        compiler_params=pltpu.CompilerParams(
            collective_id=0, vmem_limit_bytes=60 * 1024 * 1024
        ),
    )(A, B)[0]


# device time: 994451 ns/iter; 1.0001x vs baseline; 1.0001x over previous
import os

import jax

os.makedirs("/tmp/scband_jax_cache", exist_ok=True)
jax.config.update("jax_compilation_cache_dir", "/tmp/scband_jax_cache")
jax.config.update("jax_persistent_cache_min_compile_time_secs", 0)

import jax.numpy as jnp
from jax import lax
from jax.experimental import pallas as pl
from jax.experimental.pallas import tpu as pltpu

N_DEV = 4


def kernel(A, B):
    m_per, k = A.shape
    _, n = B.shape
    M = N_DEV * m_per
    half = m_per // 2

    TM = 512
    n_tiles = m_per // TM
    n_ctiles = n_tiles // 2
    n_atiles = n_tiles - n_ctiles
    piece = half // 2
    chalf = half // 2

    def body(a_ref, b_ref, out_ref, ag_ref, a_vmem, c_vmem, local_sem,
             sAR, rAR, sAL, rAL, sCR, rCR, sCL, rCL):
        my = lax.axis_index("i")
        left = (my + N_DEV - 1) % N_DEV
        right = (my + 1) % N_DEV
        diag = (my + 2) % N_DEV

        barrier = pltpu.get_barrier_semaphore()
        for nbr in (left, right):
            pl.semaphore_signal(
                barrier, inc=1, device_id=(nbr,),
                device_id_type=pl.DeviceIdType.MESH,
            )
        pl.semaphore_wait(barrier, 2)

        def rdma(src, dst, send_sem, recv_sem, dev):
            pltpu.make_async_remote_copy(
                src_ref=src, dst_ref=dst,
                send_sem=send_sem, recv_sem=recv_sem,
                device_id=(dev,), device_id_type=pl.DeviceIdType.MESH,
            ).start()

        def wait_recv(dst, recv_sem):
            pltpu.make_async_remote_copy(
                src_ref=dst, dst_ref=dst,
                send_sem=sAR.at[0], recv_sem=recv_sem,
                device_id=(right,), device_id_type=pl.DeviceIdType.MESH,
            ).wait_recv()

        def wait_send(src, send_sem):
            pltpu.make_async_remote_copy(
                src_ref=src, dst_ref=src,
                send_sem=send_sem, recv_sem=rAR.at[0],
                device_id=(right,), device_id_type=pl.DeviceIdType.MESH,
            ).wait_send()

        def matmul_tile(src_slice, out_rows):
            a_in = pltpu.make_async_copy(src_slice, a_vmem, local_sem)
            a_in.start()
            a_in.wait()
            c_vmem[...] = jnp.dot(
                a_vmem[...], b_ref[...], preferred_element_type=jnp.float32
            )
            c_out = pltpu.make_async_copy(
                c_vmem, out_ref.at[pl.ds(out_rows, TM), :], local_sem
            )
            c_out.start()
            c_out.wait()

        def own_tile(t):
            matmul_tile(a_ref.at[pl.ds(t * TM, TM), :], my * m_per + t * TM)

        def remote_tile(o, j):
            matmul_tile(
                ag_ref.at[o, pl.ds(j * TM, TM), :],
                o * m_per + half + j * TM,
            )

        def a_piece(o, p):
            return ag_ref.at[o, pl.ds(p * piece, piece), :]

        def c_msg(o, h):
            return out_ref.at[pl.ds(o * m_per + h * chalf, chalf), :]

        a_src = a_ref.at[pl.ds(half, half), :]
        rdma(a_src, ag_ref.at[my], sAR.at[0], rAR.at[0], right)
        rdma(a_src, ag_ref.at[my], sAL.at[0], rAL.at[0], left)

        def own_c_pair(h):
            lax.fori_loop(2 * h, 2 * h + 2, lambda t, _: (own_tile(t), _)[1], 0)
            rdma(c_msg(my, h), c_msg(my, h), sCR.at[h], rCR.at[h], right)
            rdma(c_msg(my, h), c_msg(my, h), sCL.at[h], rCL.at[h], left)

        own_c_pair(0)
        own_c_pair(1)

        wait_recv(ag_ref.at[left], rAR.at[0])
        rdma(a_piece(left, 0), a_piece(left, 0), sAR.at[1], rAR.at[1], right)
        wait_recv(ag_ref.at[right], rAL.at[0])
        rdma(a_piece(right, 1), a_piece(right, 1), sAL.at[1], rAL.at[1], left)

        lax.fori_loop(0, n_atiles, lambda j, _: (remote_tile(left, j), _)[1], 0)
        lax.fori_loop(0, n_atiles, lambda j, _: (remote_tile(right, j), _)[1], 0)

        wait_recv(c_msg(left, 0), rCR.at[0])
        rdma(c_msg(left, 0), c_msg(left, 0), sCR.at[2], rCR.at[2], right)

        lax.fori_loop(n_ctiles, n_tiles, lambda t, _: (own_tile(t), _)[1], 0)

        wait_recv(c_msg(right, 1), rCL.at[1])
        rdma(c_msg(right, 1), c_msg(right, 1), sCL.at[2], rCL.at[2], left)

        wait_recv(a_piece(diag, 0), rAR.at[1])
        lax.fori_loop(0, 2, lambda j, _: (remote_tile(diag, j), _)[1], 0)
        wait_recv(a_piece(diag, 1), rAL.at[1])
        lax.fori_loop(2, n_atiles, lambda j, _: (remote_tile(diag, j), _)[1], 0)

        wait_recv(c_msg(left, 1), rCR.at[1])
        wait_recv(c_msg(right, 0), rCL.at[0])
        wait_recv(c_msg(diag, 0), rCR.at[2])
        wait_recv(c_msg(diag, 1), rCL.at[2])

        wait_send(ag_ref.at[my], sAR.at[0])
        wait_send(ag_ref.at[my], sAL.at[0])
        wait_send(a_piece(my, 0), sAR.at[1])
        wait_send(a_piece(my, 0), sAL.at[1])
        for h in range(3):
            wait_send(c_msg(my, h % 2), sCR.at[h])
            wait_send(c_msg(my, h % 2), sCL.at[h])

    return pl.pallas_call(
        body,
        out_shape=[
            jax.ShapeDtypeStruct((M, n), jnp.float32),
            jax.ShapeDtypeStruct((N_DEV, half, k), jnp.float32),
        ],
        in_specs=[
            pl.BlockSpec(memory_space=pl.ANY),
            pl.BlockSpec(memory_space=pltpu.VMEM),
        ],
        out_specs=[
            pl.BlockSpec(memory_space=pl.ANY),
            pl.BlockSpec(memory_space=pl.ANY),
        ],
        scratch_shapes=[
            pltpu.VMEM((TM, k), jnp.float32),
            pltpu.VMEM((TM, n), jnp.float32),
            pltpu.SemaphoreType.DMA,
            pltpu.SemaphoreType.DMA((2,)),
            pltpu.SemaphoreType.DMA((2,)),
            pltpu.SemaphoreType.DMA((2,)),
            pltpu.SemaphoreType.DMA((2,)),
            pltpu.SemaphoreType.DMA((3,)),
            pltpu.SemaphoreType.DMA((3,)),
            pltpu.SemaphoreType.DMA((3,)),
            pltpu.SemaphoreType.DMA((3,)),
        ],
        compiler_params=pltpu.CompilerParams(
            collective_id=0, vmem_limit_bytes=60 * 1024 * 1024
        ),
    )(A, B)[0]


# device time: 859507 ns/iter; 1.1571x vs baseline; 1.1570x over previous
import os

import jax

os.makedirs("/tmp/scband_jax_cache", exist_ok=True)
jax.config.update("jax_compilation_cache_dir", "/tmp/scband_jax_cache")
jax.config.update("jax_persistent_cache_min_compile_time_secs", 0)

import jax.numpy as jnp
from jax import lax
from jax.experimental import pallas as pl
from jax.experimental.pallas import tpu as pltpu

N_DEV = 4


def kernel(A, B):
    m_per, k = A.shape
    _, n = B.shape
    M = N_DEV * m_per

    TM = 512
    n_tiles = m_per // TM
    n_ctiles = 2
    n_atiles = n_tiles - n_ctiles
    a_rows = n_atiles * TM
    piece = a_rows // 2
    c_rows = n_ctiles * TM

    def body(a_ref, b_ref, out_ref, ag_ref, a_vmem, c_vmem, aS, cS,
             sAR, rAR, sAL, rAL, sCR, rCR, sCL, rCL):
        my = lax.axis_index("i")
        left = (my + N_DEV - 1) % N_DEV
        right = (my + 1) % N_DEV
        diag = (my + 2) % N_DEV

        barrier = pltpu.get_barrier_semaphore()
        for nbr in (left, right):
            pl.semaphore_signal(
                barrier, inc=1, device_id=(nbr,),
                device_id_type=pl.DeviceIdType.MESH,
            )
        pl.semaphore_wait(barrier, 2)

        def rdma(src, dst, send_sem, recv_sem, dev):
            pltpu.make_async_remote_copy(
                src_ref=src, dst_ref=dst,
                send_sem=send_sem, recv_sem=recv_sem,
                device_id=(dev,), device_id_type=pl.DeviceIdType.MESH,
            ).start()

        def wait_recv(dst, recv_sem):
            pltpu.make_async_remote_copy(
                src_ref=dst, dst_ref=dst,
                send_sem=sAR.at[0], recv_sem=recv_sem,
                device_id=(right,), device_id_type=pl.DeviceIdType.MESH,
            ).wait_recv()

        def wait_send(src, send_sem):
            pltpu.make_async_remote_copy(
                src_ref=src, dst_ref=src,
                send_sem=send_sem, recv_sem=rAR.at[0],
                device_id=(right,), device_id_type=pl.DeviceIdType.MESH,
            ).wait_send()

        def pipeline(r, src_fn, dst_row_fn):
            def a_copy(t, slot):
                return pltpu.make_async_copy(
                    src_fn(t), a_vmem.at[slot], aS.at[slot]
                )

            def c_copy(t, slot):
                return pltpu.make_async_copy(
                    c_vmem.at[slot],
                    out_ref.at[pl.ds(dst_row_fn(t), TM), :],
                    cS.at[slot],
                )

            a_copy(0, 0).start()

            def step(t, carry):
                slot = t % 2

                @pl.when(t + 1 < r)
                def _():
                    a_copy(t + 1, (t + 1) % 2).start()

                a_copy(t, slot).wait()

                @pl.when(t >= 2)
                def _():
                    c_copy(t - 2, slot).wait()

                c_vmem[slot] = jnp.dot(
                    a_vmem[slot], b_ref[...],
                    preferred_element_type=jnp.float32,
                )
                c_copy(t, slot).start()
                return carry

            lax.fori_loop(0, r, step, 0)
            if r >= 2:
                c_copy(r - 2, (r - 2) % 2).wait()
            c_copy(r - 1, (r - 1) % 2).wait()

        def own_src(t):
            return a_ref.at[pl.ds(t * TM, TM), :]

        def remote_src(o):
            return lambda t: ag_ref.at[o, pl.ds(t * TM, TM), :]

        def a_fwd(o, p):
            return ag_ref.at[o, pl.ds(p * piece, piece), :]

        def c_msg(o):
            return out_ref.at[pl.ds(o * m_per, c_rows), :]

        def c_fwd(o, t):
            return out_ref.at[pl.ds(o * m_per + t * TM, TM), :]

        a_src = a_ref.at[pl.ds(c_rows, a_rows), :]
        rdma(a_src, ag_ref.at[my], sAR.at[0], rAR.at[0], right)
        rdma(a_src, ag_ref.at[my], sAL.at[0], rAL.at[0], left)

        pipeline(n_ctiles, own_src, lambda t: my * m_per + t * TM)
        rdma(c_msg(my), c_msg(my), sCR.at[0], rCR.at[0], right)
        rdma(c_msg(my), c_msg(my), sCL.at[0], rCL.at[0], left)

        pipeline(
            n_atiles,
            lambda t: a_ref.at[pl.ds(c_rows + t * TM, TM), :],
            lambda t: my * m_per + c_rows + t * TM,
        )

        wait_recv(ag_ref.at[left], rAR.at[0])
        rdma(a_fwd(left, 0), a_fwd(left, 0), sAR.at[1], rAR.at[1], right)
        wait_recv(ag_ref.at[right], rAL.at[0])
        rdma(a_fwd(right, 1), a_fwd(right, 1), sAL.at[1], rAL.at[1], left)

        pipeline(n_atiles, remote_src(left),
                 lambda t: left * m_per + c_rows + t * TM)
        pipeline(n_atiles, remote_src(right),
                 lambda t: right * m_per + c_rows + t * TM)

        wait_recv(c_msg(left), rCR.at[0])
        rdma(c_fwd(left, 0), c_fwd(left, 0), sCR.at[1], rCR.at[1], right)
        wait_recv(c_msg(right), rCL.at[0])
        rdma(c_fwd(right, 1), c_fwd(right, 1), sCL.at[1], rCL.at[1], left)

        wait_recv(a_fwd(diag, 0), rAR.at[1])
        pipeline(n_atiles // 2, remote_src(diag),
                 lambda t: diag * m_per + c_rows + t * TM)
        wait_recv(a_fwd(diag, 1), rAL.at[1])
        pipeline(n_atiles // 2,
                 lambda t: ag_ref.at[diag, pl.ds(piece + t * TM, TM), :],
                 lambda t: diag * m_per + c_rows + piece + t * TM)

        wait_recv(c_fwd(diag, 0), rCR.at[1])
        wait_recv(c_fwd(diag, 1), rCL.at[1])

        wait_send(a_src, sAR.at[0])
        wait_send(a_src, sAL.at[0])
        wait_send(a_fwd(my, 0), sAR.at[1])
        wait_send(a_fwd(my, 0), sAL.at[1])
        wait_send(c_msg(my), sCR.at[0])
        wait_send(c_msg(my), sCL.at[0])
        wait_send(c_fwd(my, 0), sCR.at[1])
        wait_send(c_fwd(my, 0), sCL.at[1])

    return pl.pallas_call(
        body,
        out_shape=[
            jax.ShapeDtypeStruct((M, n), jnp.float32),
            jax.ShapeDtypeStruct((N_DEV, a_rows, k), jnp.float32),
        ],
        in_specs=[
            pl.BlockSpec(memory_space=pl.ANY),
            pl.BlockSpec(memory_space=pltpu.VMEM),
        ],
        out_specs=[
            pl.BlockSpec(memory_space=pl.ANY),
            pl.BlockSpec(memory_space=pl.ANY),
        ],
        scratch_shapes=[
            pltpu.VMEM((2, TM, k), jnp.float32),
            pltpu.VMEM((2, TM, n), jnp.float32),
            pltpu.SemaphoreType.DMA((2,)),
            pltpu.SemaphoreType.DMA((2,)),
            pltpu.SemaphoreType.DMA((2,)),
            pltpu.SemaphoreType.DMA((2,)),
            pltpu.SemaphoreType.DMA((2,)),
            pltpu.SemaphoreType.DMA((2,)),
            pltpu.SemaphoreType.DMA((2,)),
            pltpu.SemaphoreType.DMA((2,)),
            pltpu.SemaphoreType.DMA((2,)),
            pltpu.SemaphoreType.DMA((2,)),
        ],
        compiler_params=pltpu.CompilerParams(
            collective_id=0, vmem_limit_bytes=60 * 1024 * 1024
        ),
    )(A, B)[0]


# device time: 772572 ns/iter; 1.2873x vs baseline; 1.1125x over previous
import os

import jax

os.makedirs("/tmp/scband_jax_cache", exist_ok=True)
jax.config.update("jax_compilation_cache_dir", "/tmp/scband_jax_cache")
jax.config.update("jax_persistent_cache_min_compile_time_secs", 0)

import jax.numpy as jnp
from jax import lax
from jax.experimental import pallas as pl
from jax.experimental.pallas import tpu as pltpu

N_DEV = 4


def kernel(A, B):
    m_per, k = A.shape
    _, n = B.shape
    M = N_DEV * m_per
    half = m_per // 2
    quart = half // 2

    TM = 512
    n_tiles = m_per // TM
    tiles_half = half // TM
    tiles_quart = quart // TM

    def body(a_ref, b_ref, out_ref, ag_ref, a_vmem, c_vmem, aS, cS,
             sAR, rAR, sAL, rAL):
        my = lax.axis_index("i")
        left = (my + N_DEV - 1) % N_DEV
        right = (my + 1) % N_DEV
        diag = (my + 2) % N_DEV

        barrier = pltpu.get_barrier_semaphore()
        for nbr in (left, right):
            pl.semaphore_signal(
                barrier, inc=1, device_id=(nbr,),
                device_id_type=pl.DeviceIdType.MESH,
            )
        pl.semaphore_wait(barrier, 2)

        def rdma(src, dst, send_sem, recv_sem, dev):
            pltpu.make_async_remote_copy(
                src_ref=src, dst_ref=dst,
                send_sem=send_sem, recv_sem=recv_sem,
                device_id=(dev,), device_id_type=pl.DeviceIdType.MESH,
            ).start()

        def wait_recv(dst, recv_sem):
            pltpu.make_async_remote_copy(
                src_ref=dst, dst_ref=dst,
                send_sem=sAR.at[0], recv_sem=recv_sem,
                device_id=(right,), device_id_type=pl.DeviceIdType.MESH,
            ).wait_recv()

        def wait_send(src, send_sem):
            pltpu.make_async_remote_copy(
                src_ref=src, dst_ref=src,
                send_sem=send_sem, recv_sem=rAR.at[0],
                device_id=(right,), device_id_type=pl.DeviceIdType.MESH,
            ).wait_send()

        def pipeline(r, src_fn, dst_row_fn):
            def a_copy(t, slot):
                return pltpu.make_async_copy(
                    src_fn(t), a_vmem.at[slot], aS.at[slot]
                )

            def c_copy(t, slot):
                return pltpu.make_async_copy(
                    c_vmem.at[slot],
                    out_ref.at[pl.ds(dst_row_fn(t), TM), :],
                    cS.at[slot],
                )

            a_copy(0, 0).start()

            def step(t, carry):
                slot = t % 2

                @pl.when(t + 1 < r)
                def _():
                    a_copy(t + 1, (t + 1) % 2).start()

                a_copy(t, slot).wait()

                @pl.when(t >= 2)
                def _():
                    c_copy(t - 2, slot).wait()

                c_vmem[slot] = jnp.dot(
                    a_vmem[slot], b_ref[...],
                    preferred_element_type=jnp.float32,
                )
                c_copy(t, slot).start()
                return carry

            lax.fori_loop(0, r, step, 0)
            if r >= 2:
                c_copy(r - 2, (r - 2) % 2).wait()
            c_copy(r - 1, (r - 1) % 2).wait()

        def chunk_tiles(o, j0):
            return (
                lambda t: ag_ref.at[o, pl.ds(j0 * TM + t * TM, TM), :],
                lambda t: o * m_per + j0 * TM + t * TM,
            )

        def a_rows(o, start, nrows):
            return ag_ref.at[o, pl.ds(start, nrows), :]

        p0 = a_ref.at[pl.ds(0, half), :]
        p1 = a_ref.at[pl.ds(half, half), :]
        rdma(p0, a_rows(my, 0, half), sAR.at[0], rAR.at[0], right)
        rdma(p1, a_rows(my, half, half), sAR.at[1], rAR.at[1], right)
        rdma(p1, a_rows(my, half, half), sAL.at[0], rAL.at[0], left)
        rdma(p0, a_rows(my, 0, half), sAL.at[1], rAL.at[1], left)

        pipeline(n_tiles,
                 lambda t: a_ref.at[pl.ds(t * TM, TM), :],
                 lambda t: my * m_per + t * TM)

        wait_recv(a_rows(left, 0, half), rAR.at[0])
        for q in range(2):
            rdma(a_rows(left, q * quart, quart),
                 a_rows(left, q * quart, quart),
                 sAR.at[2 + q], rAR.at[2 + q], right)
        pipeline(tiles_half, *chunk_tiles(left, 0))

        wait_recv(a_rows(right, half, half), rAL.at[0])
        for q in range(2):
            rdma(a_rows(right, half + q * quart, quart),
                 a_rows(right, half + q * quart, quart),
                 sAL.at[2 + q], rAL.at[2 + q], left)
        pipeline(tiles_half, *chunk_tiles(right, tiles_half))

        wait_recv(a_rows(left, half, half), rAR.at[1])
        pipeline(tiles_half, *chunk_tiles(left, tiles_half))
        wait_recv(a_rows(right, 0, half), rAL.at[1])
        pipeline(tiles_half, *chunk_tiles(right, 0))

        wait_recv(a_rows(diag, 0, quart), rAR.at[2])
        pipeline(tiles_quart, *chunk_tiles(diag, 0))
        wait_recv(a_rows(diag, half, quart), rAL.at[2])
        pipeline(tiles_quart, *chunk_tiles(diag, tiles_half))
        wait_recv(a_rows(diag, quart, quart), rAR.at[3])
        pipeline(tiles_quart, *chunk_tiles(diag, tiles_quart))
        wait_recv(a_rows(diag, half + quart, quart), rAL.at[3])
        pipeline(tiles_quart, *chunk_tiles(diag, tiles_half + tiles_quart))

        wait_send(p0, sAR.at[0])
        wait_send(p1, sAR.at[1])
        wait_send(p1, sAL.at[0])
        wait_send(p0, sAL.at[1])
        for q in range(2):
            wait_send(a_rows(my, q * quart, quart), sAR.at[2 + q])
            wait_send(a_rows(my, q * quart, quart), sAL.at[2 + q])

    return pl.pallas_call(
        body,
        out_shape=[
            jax.ShapeDtypeStruct((M, n), jnp.float32),
            jax.ShapeDtypeStruct((N_DEV, m_per, k), jnp.float32),
        ],
        in_specs=[
            pl.BlockSpec(memory_space=pl.ANY),
            pl.BlockSpec(memory_space=pltpu.VMEM),
        ],
        out_specs=[
            pl.BlockSpec(memory_space=pl.ANY),
            pl.BlockSpec(memory_space=pl.ANY),
        ],
        scratch_shapes=[
            pltpu.VMEM((2, TM, k), jnp.float32),
            pltpu.VMEM((2, TM, n), jnp.float32),
            pltpu.SemaphoreType.DMA((2,)),
            pltpu.SemaphoreType.DMA((2,)),
            pltpu.SemaphoreType.DMA((4,)),
            pltpu.SemaphoreType.DMA((4,)),
            pltpu.SemaphoreType.DMA((4,)),
            pltpu.SemaphoreType.DMA((4,)),
        ],
        compiler_params=pltpu.CompilerParams(
            collective_id=0, vmem_limit_bytes=60 * 1024 * 1024
        ),
    )(A, B)[0]
